# Initial kernel scaffold; baseline (speedup 1.0000x reference)
import jax
import jax.numpy as jnp
from jax.experimental import pallas as pl


def kernel(x, conditions, float_masks, style_w, in_b, in_w, rs_w, rs_b):
    raise NotImplementedError("write your pallas kernel here")


# speedup vs baseline: 2.5717x; 2.5717x over previous
import functools

import jax
import jax.numpy as jnp
from jax.experimental import pallas as pl
from jax.experimental.pallas import tpu as pltpu


def _wavenet_body(x_ref, c_ref, m_ref,
                  wt_ref, ws_ref, stt_ref, sts_ref, bt_ref, bs_ref,
                  rsr_ref, rsk_ref, rbr_ref, rbk_ref,
                  o_ref, *, seqs, chans, stack, taps, dilation_rate):
    G, C, S, K = seqs, chans, stack, taps
    R = G * C
    T = x_ref.shape[-1]

    xv = x_ref[...]
    cv = c_ref[...]
    mask_b = jnp.broadcast_to(m_ref[...], (G, C, T)).reshape(R, T)
    lane = jax.lax.broadcasted_iota(jnp.int32, (R, T), 1)

    skip = jnp.zeros((R, T), jnp.float32)
    for i in range(S):
        d = dilation_rate ** i
        sb_t = (jnp.dot(stt_ref[i], cv, preferred_element_type=jnp.float32)
                + bt_ref[i])
        sb_s = (jnp.dot(sts_ref[i], cv, preferred_element_type=jnp.float32)
                + bs_ref[i])
        tm = jnp.where(lane >= d, pltpu.roll(xv, d, axis=1), 0.0)
        tp = jnp.where(lane < T - d, pltpu.roll(xv, T - d, axis=1), 0.0)
        z_t = (sb_t
               + jnp.dot(wt_ref[i, 0], tm, preferred_element_type=jnp.float32)
               + jnp.dot(wt_ref[i, 1], xv, preferred_element_type=jnp.float32)
               + jnp.dot(wt_ref[i, 2], tp, preferred_element_type=jnp.float32))
        z_s = (sb_s
               + jnp.dot(ws_ref[i, 0], tm, preferred_element_type=jnp.float32)
               + jnp.dot(ws_ref[i, 1], xv, preferred_element_type=jnp.float32)
               + jnp.dot(ws_ref[i, 2], tp, preferred_element_type=jnp.float32))
        acts = jnp.tanh(z_t) * jax.nn.sigmoid(z_s)
        res = (jnp.dot(rsr_ref[i], acts, preferred_element_type=jnp.float32)
               + rbr_ref[i])
        skip = (skip
                + jnp.dot(rsk_ref[i], acts, preferred_element_type=jnp.float32)
                + rbk_ref[i])
        xv = (xv + res) * mask_b

    o_ref[...] = (skip * mask_b).astype(o_ref.dtype)


def kernel(x, conditions, float_masks, style_w, in_b, in_w, rs_w, rs_b):
    B, C, T = x.shape
    Cs = conditions.shape[1]
    S, K = in_w.shape[0], in_w.shape[1]
    dilation_rate = 2

    G = 32
    while B % G:
        G //= 2
    R = G * C
    num_blocks = B // G

    x2 = x.reshape(B * C, T)
    c2 = conditions.reshape(B * Cs, T)

    eye_g = jnp.eye(G, dtype=jnp.float32)

    def bd(w):
        out = jnp.einsum('gh,...ac->...gahc', eye_g, w.astype(jnp.float32))
        return out.reshape(*w.shape[:-2], G * w.shape[-2], G * w.shape[-1])

    wt = bd(in_w[:, :, :C, :])
    ws = bd(in_w[:, :, C:, :])
    sw = style_w.reshape(S, 2 * C, Cs)
    stt = bd(sw[:, :C, :])
    sts = bd(sw[:, C:, :])
    rsr = bd(rs_w[:, :C, :])
    rsk = bd(rs_w[:, C:, :])
    bt = jnp.tile(in_b[:, :C, :], (1, G, 1))
    bs = jnp.tile(in_b[:, C:, :], (1, G, 1))
    rbr = jnp.tile(rs_b[:, :C, :], (1, G, 1))
    rbk = jnp.tile(rs_b[:, C:, :], (1, G, 1))

    body = functools.partial(
        _wavenet_body, seqs=G, chans=C, stack=S, taps=K,
        dilation_rate=dilation_rate)

    const = lambda *shape: (shape, lambda b: (0,) * len(shape))

    out = pl.pallas_call(
        body,
        out_shape=jax.ShapeDtypeStruct((B * C, T), jnp.float32),
        grid=(num_blocks,),
        in_specs=[
            pl.BlockSpec((R, T), lambda b: (b, 0)),
            pl.BlockSpec((R, T), lambda b: (b, 0)),
            pl.BlockSpec((G, 1, T), lambda b: (b, 0, 0)),
            pl.BlockSpec(*const(S, K, R, R)),
            pl.BlockSpec(*const(S, K, R, R)),
            pl.BlockSpec(*const(S, R, G * Cs)),
            pl.BlockSpec(*const(S, R, G * Cs)),
            pl.BlockSpec(*const(S, R, 1)),
            pl.BlockSpec(*const(S, R, 1)),
            pl.BlockSpec(*const(S, R, R)),
            pl.BlockSpec(*const(S, R, R)),
            pl.BlockSpec(*const(S, R, 1)),
            pl.BlockSpec(*const(S, R, 1)),
        ],
        out_specs=pl.BlockSpec((R, T), lambda b: (b, 0)),
        compiler_params=pltpu.CompilerParams(
            dimension_semantics=("parallel",),
            vmem_limit_bytes=56 * 1024 * 1024),
    )(x2, c2, float_masks, wt, ws, stt, sts, bt, bs, rsr, rsk, rbr, rbk)

    return out.reshape(B, C, T)


# speedup vs baseline: 5.8472x; 2.2737x over previous
import functools

import jax
import jax.numpy as jnp
from jax.experimental import pallas as pl
from jax.experimental.pallas import tpu as pltpu


def _prep_body(in_w_ref, sw_ref, rs_w_ref, in_b_ref, rs_b_ref,
               wt_ref, ws_ref, stt_ref, sts_ref,
               rsr_ref, rsk_ref, bt_ref, bs_ref, rbr_ref, rbk_ref,
               *, seqs, chans, stack, taps):
    G, C, S, K = seqs, chans, stack, taps
    R = G * C
    p_row = jax.lax.broadcasted_iota(jnp.int32, (R, C), 0) % C
    p_col = jax.lax.broadcasted_iota(jnp.int32, (R, C), 1)
    P = (p_row == p_col).astype(jnp.float32)
    q_row = jax.lax.broadcasted_iota(jnp.int32, (C, R), 0)
    q_col = jax.lax.broadcasted_iota(jnp.int32, (C, R), 1) % C
    Q = (q_row == q_col).astype(jnp.float32)
    blk = (jax.lax.broadcasted_iota(jnp.int32, (R, R), 0) // C ==
           jax.lax.broadcasted_iota(jnp.int32, (R, R), 1) // C)

    def bd(w):
        tiled = jnp.dot(jnp.dot(P, w, preferred_element_type=jnp.float32), Q,
                        preferred_element_type=jnp.float32)
        return jnp.where(blk, tiled, 0.0)

    def tile_b(b):
        return jnp.dot(P, b, preferred_element_type=jnp.float32)

    for i in range(S):
        for k in range(K):
            wt_ref[i, k] = bd(in_w_ref[i, k, :C, :])
            ws_ref[i, k] = bd(in_w_ref[i, k, C:, :])
        stt_ref[i] = bd(sw_ref[i, :C, :])
        sts_ref[i] = bd(sw_ref[i, C:, :])
        rsr_ref[i] = bd(rs_w_ref[i, :C, :])
        rsk_ref[i] = bd(rs_w_ref[i, C:, :])
        bt_ref[i] = tile_b(in_b_ref[i, :C, :])
        bs_ref[i] = tile_b(in_b_ref[i, C:, :])
        rbr_ref[i] = tile_b(rs_b_ref[i, :C, :])
        rbk_ref[i] = tile_b(rs_b_ref[i, C:, :])


def _wavenet_body(x_ref, c_ref, m_ref,
                  wt_ref, ws_ref, stt_ref, sts_ref, bt_ref, bs_ref,
                  rsr_ref, rsk_ref, rbr_ref, rbk_ref,
                  o_ref, *, seqs, chans, stack, dilation_rate):
    G, C, S = seqs, chans, stack
    R = G * C
    T = x_ref.shape[-1]

    xv = x_ref[...]
    cv = c_ref[...]
    mask_b = jnp.broadcast_to(m_ref[...], (G, C, T)).reshape(R, T)
    lane = jax.lax.broadcasted_iota(jnp.int32, (R, T), 1)

    skip = jnp.zeros((R, T), jnp.float32)
    for i in range(S):
        d = dilation_rate ** i
        sb_t = (jnp.dot(stt_ref[i], cv, preferred_element_type=jnp.float32)
                + bt_ref[i])
        sb_s = (jnp.dot(sts_ref[i], cv, preferred_element_type=jnp.float32)
                + bs_ref[i])
        tm = jnp.where(lane >= d, pltpu.roll(xv, d, axis=1), 0.0)
        tp = jnp.where(lane < T - d, pltpu.roll(xv, T - d, axis=1), 0.0)
        z_t = (sb_t
               + jnp.dot(wt_ref[i, 0], tm, preferred_element_type=jnp.float32)
               + jnp.dot(wt_ref[i, 1], xv, preferred_element_type=jnp.float32)
               + jnp.dot(wt_ref[i, 2], tp, preferred_element_type=jnp.float32))
        z_s = (sb_s
               + jnp.dot(ws_ref[i, 0], tm, preferred_element_type=jnp.float32)
               + jnp.dot(ws_ref[i, 1], xv, preferred_element_type=jnp.float32)
               + jnp.dot(ws_ref[i, 2], tp, preferred_element_type=jnp.float32))
        acts = jnp.tanh(z_t) * jax.nn.sigmoid(z_s)
        res = (jnp.dot(rsr_ref[i], acts, preferred_element_type=jnp.float32)
               + rbr_ref[i])
        skip = (skip
                + jnp.dot(rsk_ref[i], acts, preferred_element_type=jnp.float32)
                + rbk_ref[i])
        xv = (xv + res) * mask_b

    o_ref[...] = (skip * mask_b).astype(o_ref.dtype)


def kernel(x, conditions, float_masks, style_w, in_b, in_w, rs_w, rs_b):
    B, C, T = x.shape
    Cs = conditions.shape[1]
    S, K = in_w.shape[0], in_w.shape[1]
    assert Cs == C
    dilation_rate = 2

    G = 32
    while B % G:
        G //= 2
    R = G * C
    num_blocks = B // G

    x2 = x.reshape(B * C, T)
    c2 = conditions.reshape(B * Cs, T)
    sw = style_w.reshape(S, 2 * C, Cs)

    prep = pl.pallas_call(
        functools.partial(_prep_body, seqs=G, chans=C, stack=S, taps=K),
        out_shape=[
            jax.ShapeDtypeStruct((S, K, R, R), jnp.float32),
            jax.ShapeDtypeStruct((S, K, R, R), jnp.float32),
            jax.ShapeDtypeStruct((S, R, R), jnp.float32),
            jax.ShapeDtypeStruct((S, R, R), jnp.float32),
            jax.ShapeDtypeStruct((S, R, R), jnp.float32),
            jax.ShapeDtypeStruct((S, R, R), jnp.float32),
            jax.ShapeDtypeStruct((S, R, 1), jnp.float32),
            jax.ShapeDtypeStruct((S, R, 1), jnp.float32),
            jax.ShapeDtypeStruct((S, R, 1), jnp.float32),
            jax.ShapeDtypeStruct((S, R, 1), jnp.float32),
        ],
    )
    wt, ws, stt, sts, rsr, rsk, bt, bs, rbr, rbk = prep(
        in_w, sw, rs_w, in_b, rs_b)

    body = functools.partial(
        _wavenet_body, seqs=G, chans=C, stack=S,
        dilation_rate=dilation_rate)

    const = lambda *shape: (shape, lambda b: (0,) * len(shape))

    out = pl.pallas_call(
        body,
        out_shape=jax.ShapeDtypeStruct((B * C, T), jnp.float32),
        grid=(num_blocks,),
        in_specs=[
            pl.BlockSpec((R, T), lambda b: (b, 0)),
            pl.BlockSpec((R, T), lambda b: (b, 0)),
            pl.BlockSpec((G, 1, T), lambda b: (b, 0, 0)),
            pl.BlockSpec(*const(S, K, R, R)),
            pl.BlockSpec(*const(S, K, R, R)),
            pl.BlockSpec(*const(S, R, R)),
            pl.BlockSpec(*const(S, R, R)),
            pl.BlockSpec(*const(S, R, 1)),
            pl.BlockSpec(*const(S, R, 1)),
            pl.BlockSpec(*const(S, R, R)),
            pl.BlockSpec(*const(S, R, R)),
            pl.BlockSpec(*const(S, R, 1)),
            pl.BlockSpec(*const(S, R, 1)),
        ],
        out_specs=pl.BlockSpec((R, T), lambda b: (b, 0)),
        compiler_params=pltpu.CompilerParams(
            dimension_semantics=("parallel",),
            vmem_limit_bytes=56 * 1024 * 1024),
    )(x2, c2, float_masks, wt, ws, stt, sts, bt, bs, rsr, rsk, rbr, rbk)

    return out.reshape(B, C, T)


# speedup vs baseline: 6.2813x; 1.0742x over previous
import functools

import jax
import jax.numpy as jnp
from jax.experimental import pallas as pl
from jax.experimental.pallas import tpu as pltpu


def _body(x_ref, c_ref, m_ref,
          in_w_ref, sw_ref, rs_w_ref, in_b_ref, rs_b_ref,
          o_ref,
          wt_s, ws_s, stt_s, sts_s, rsr_s, rsk_s, b_s,
          *, seqs, chans, stack, taps, dilation_rate):
    G, C, S, K = seqs, chans, stack, taps
    R = G * C
    T = x_ref.shape[-1]

    @pl.when(pl.program_id(0) == 0)
    def _prep():
        p_row = jax.lax.broadcasted_iota(jnp.int32, (R, C), 0) % C
        p_col = jax.lax.broadcasted_iota(jnp.int32, (R, C), 1)
        P = (p_row == p_col).astype(jnp.float32)
        q_row = jax.lax.broadcasted_iota(jnp.int32, (C, R), 0)
        q_col = jax.lax.broadcasted_iota(jnp.int32, (C, R), 1) % C
        Q = (q_row == q_col).astype(jnp.float32)
        blk = (jax.lax.broadcasted_iota(jnp.int32, (R, R), 0) // C ==
               jax.lax.broadcasted_iota(jnp.int32, (R, R), 1) // C)

        def bd(w):
            tiled = jnp.dot(jnp.dot(P, w, preferred_element_type=jnp.float32),
                            Q, preferred_element_type=jnp.float32)
            return jnp.where(blk, tiled, 0.0)

        for i in range(S):
            for k in range(K):
                wt_s[i, k] = bd(in_w_ref[i, k, :C, :])
                ws_s[i, k] = bd(in_w_ref[i, k, C:, :])
            stt_s[i] = bd(sw_ref[i, :C, :])
            sts_s[i] = bd(sw_ref[i, C:, :])
            rsr_s[i] = bd(rs_w_ref[i, :C, :])
            rsk_s[i] = bd(rs_w_ref[i, C:, :])
            b_s[i, 0] = jnp.dot(P, in_b_ref[i, :C, :],
                                preferred_element_type=jnp.float32)
            b_s[i, 1] = jnp.dot(P, in_b_ref[i, C:, :],
                                preferred_element_type=jnp.float32)
            b_s[i, 2] = jnp.dot(P, rs_b_ref[i, :C, :],
                                preferred_element_type=jnp.float32)
            b_s[i, 3] = jnp.dot(P, rs_b_ref[i, C:, :],
                                preferred_element_type=jnp.float32)

    xv = x_ref[...]
    cv = c_ref[...]
    mask_b = jnp.broadcast_to(m_ref[...], (G, C, T)).reshape(R, T)
    lane = jax.lax.broadcasted_iota(jnp.int32, (R, T), 1)

    skip = jnp.zeros((R, T), jnp.float32)
    for i in range(S):
        d = dilation_rate ** i
        sb_t = (jnp.dot(stt_s[i], cv, preferred_element_type=jnp.float32)
                + b_s[i, 0])
        sb_s = (jnp.dot(sts_s[i], cv, preferred_element_type=jnp.float32)
                + b_s[i, 1])
        tm = jnp.where(lane >= d, pltpu.roll(xv, d, axis=1), 0.0)
        tp = jnp.where(lane < T - d, pltpu.roll(xv, T - d, axis=1), 0.0)
        z_t = (sb_t
               + jnp.dot(wt_s[i, 0], tm, preferred_element_type=jnp.float32)
               + jnp.dot(wt_s[i, 1], xv, preferred_element_type=jnp.float32)
               + jnp.dot(wt_s[i, 2], tp, preferred_element_type=jnp.float32))
        z_s = (sb_s
               + jnp.dot(ws_s[i, 0], tm, preferred_element_type=jnp.float32)
               + jnp.dot(ws_s[i, 1], xv, preferred_element_type=jnp.float32)
               + jnp.dot(ws_s[i, 2], tp, preferred_element_type=jnp.float32))
        acts = jnp.tanh(z_t) * jax.nn.sigmoid(z_s)
        res = (jnp.dot(rsr_s[i], acts, preferred_element_type=jnp.float32)
               + b_s[i, 2])
        skip = (skip
                + jnp.dot(rsk_s[i], acts, preferred_element_type=jnp.float32)
                + b_s[i, 3])
        xv = (xv + res) * mask_b

    o_ref[...] = (skip * mask_b).astype(o_ref.dtype)


def kernel(x, conditions, float_masks, style_w, in_b, in_w, rs_w, rs_b):
    B, C, T = x.shape
    Cs = conditions.shape[1]
    S, K = in_w.shape[0], in_w.shape[1]
    assert Cs == C
    dilation_rate = 2

    G = 32
    while B % G:
        G //= 2
    R = G * C
    num_blocks = B // G

    x2 = x.reshape(B * C, T)
    c2 = conditions.reshape(B * Cs, T)
    sw = style_w.reshape(S, 2 * C, Cs)

    body = functools.partial(
        _body, seqs=G, chans=C, stack=S, taps=K,
        dilation_rate=dilation_rate)

    const = lambda *shape: (shape, lambda b: (0,) * len(shape))

    out = pl.pallas_call(
        body,
        out_shape=jax.ShapeDtypeStruct((B * C, T), jnp.float32),
        grid=(num_blocks,),
        in_specs=[
            pl.BlockSpec((R, T), lambda b: (b, 0)),
            pl.BlockSpec((R, T), lambda b: (b, 0)),
            pl.BlockSpec((G, 1, T), lambda b: (b, 0, 0)),
            pl.BlockSpec(*const(S, K, 2 * C, C)),
            pl.BlockSpec(*const(S, 2 * C, Cs)),
            pl.BlockSpec(*const(S, 2 * C, C)),
            pl.BlockSpec(*const(S, 2 * C, 1)),
            pl.BlockSpec(*const(S, 2 * C, 1)),
        ],
        out_specs=pl.BlockSpec((R, T), lambda b: (b, 0)),
        scratch_shapes=[
            pltpu.VMEM((S, K, R, R), jnp.float32),
            pltpu.VMEM((S, K, R, R), jnp.float32),
            pltpu.VMEM((S, R, R), jnp.float32),
            pltpu.VMEM((S, R, R), jnp.float32),
            pltpu.VMEM((S, R, R), jnp.float32),
            pltpu.VMEM((S, R, R), jnp.float32),
            pltpu.VMEM((S, 4, R, 1), jnp.float32),
        ],
        compiler_params=pltpu.CompilerParams(
            dimension_semantics=("arbitrary",),
            vmem_limit_bytes=56 * 1024 * 1024),
    )(x2, c2, float_masks, in_w, sw, rs_w, in_b, rs_b)

    return out.reshape(B, C, T)
